# Initial kernel scaffold; baseline (speedup 1.0000x reference)
import functools

import jax
import jax.numpy as jnp
from jax import lax
from jax.experimental import pallas as pl
from jax.experimental.pallas import tpu as pltpu
from jax.experimental.pallas import tpu_sc as plsc

N = 50000
E = 800000
NLBL = 100000
DIN, D1, D2 = 500, 128, 64

NC, NS = 2, 16
NW = NC * NS

EB = 128


def _cdiv(a, b):
    return (a + b - 1) // b


def _rup(a, b):
    return _cdiv(a, b) * b


def _make_deg(n, erows_pad, db):
    acc_rows = n + 8
    rng8 = _rup(_cdiv(n, NS), 8)
    rows_w = erows_pad // NW
    nblk = rows_w // db
    zlen = rng8 + 8

    def body(dst_hbm, out_hbm, dacc, dbuf, ones_v, zbuf):
        c = lax.axis_index("c")
        s = lax.axis_index("s")
        w = c * NS + s

        @pl.loop(0, 8)
        def _(k):
            ones_v[pl.ds(k * 16, 16)] = jnp.ones((16,), jnp.float32)

        @pl.loop(0, zlen // 16)
        def _(k):
            zbuf[pl.ds(k * 16, 16)] = jnp.zeros((16,), jnp.float32)

        start = s * rng8
        last_len = n + 8 - (NS - 1) * rng8

        @pl.when(s < NS - 1)
        def _():
            pltpu.sync_copy(zbuf.at[pl.ds(0, rng8)], dacc.at[pl.ds(start, rng8)])

        @pl.when(s == NS - 1)
        def _():
            pltpu.sync_copy(zbuf.at[pl.ds(0, last_len)], dacc.at[pl.ds(start, last_len)])

        plsc.subcore_barrier()

        @pl.loop(0, nblk)
        def _(g):
            pltpu.sync_copy(dst_hbm.at[pl.ds(w * rows_w + g * db, db)], dbuf)
            for j in range(db):
                pltpu.sync_copy(ones_v, dacc.at[dbuf.at[j]], add=True)

        plsc.subcore_barrier()

        drain_len = min(rng8, n - (NS - 1) * rng8)

        @pl.when(s < NS - 1)
        def _():
            pltpu.sync_copy(dacc.at[pl.ds(start, rng8)],
                            out_hbm.at[c].at[pl.ds(start, rng8)])

        @pl.when(s == NS - 1)
        def _():
            pltpu.sync_copy(dacc.at[pl.ds(start, drain_len)],
                            out_hbm.at[c].at[pl.ds(start, drain_len)])

    return pl.kernel(
        body,
        out_type=jax.ShapeDtypeStruct((NC, n), jnp.float32),
        mesh=plsc.VectorSubcoreMesh(core_axis_name="c", subcore_axis_name="s",
                                    num_cores=NC, num_subcores=NS),
        scratch_types=[
            pltpu.VMEM_SHARED((acc_rows,), jnp.float32),
            pltpu.VMEM((db, EB), jnp.int32),
            pltpu.VMEM((EB,), jnp.float32),
            pltpu.VMEM((zlen,), jnp.float32),
        ],
    )


def _make_agg(n, erows_pad, nchunk, blk, zrows):
    acc_rows = n + 8
    nr = nchunk // NC
    rows_t = erows_pad // NS
    nblk = rows_t // blk
    nz = (n // NS) // zrows

    def body(table, src_hbm, dst_hbm, out_hbm, acc, sbuf, dbuf, obuf, rbuf, zbuf):
        c = lax.axis_index("c")
        s = lax.axis_index("s")

        @pl.loop(0, zrows)
        def _(i):
            zbuf[i, pl.ds(0, 16)] = jnp.zeros((16,), jnp.float32)
            zbuf[i, pl.ds(16, 16)] = jnp.zeros((16,), jnp.float32)

        for r in range(nr):
            chunk = c * nr + r
            off = chunk * n

            @pl.loop(0, nz)
            def _(k):
                pltpu.sync_copy(zbuf, acc.at[pl.ds(s * (n // NS) + k * zrows, zrows)])

            @pl.when(s == 0)
            def _():
                pltpu.sync_copy(zbuf.at[pl.ds(0, 8)], acc.at[pl.ds(n, 8)])

            plsc.subcore_barrier()

            @pl.loop(0, nblk)
            def _(g):
                b0 = s * rows_t + g * blk
                pltpu.sync_copy(src_hbm.at[pl.ds(b0, blk)], sbuf)
                pltpu.sync_copy(dst_hbm.at[pl.ds(b0, blk)], dbuf)
                for j in range(blk):
                    for k in range(EB // 16):
                        obuf[j, pl.ds(k * 16, 16)] = sbuf[j, pl.ds(k * 16, 16)] + off
                for j in range(blk):
                    pltpu.sync_copy(table.at[obuf.at[j]], rbuf.at[j])
                    pltpu.sync_copy(rbuf.at[j], acc.at[dbuf.at[j]], add=True)

            plsc.subcore_barrier()

            @pl.loop(0, nz)
            def _(k):
                r0 = s * (n // NS) + k * zrows
                pltpu.sync_copy(acc.at[pl.ds(r0, zrows)],
                                out_hbm.at[pl.ds(off + r0, zrows)])

            plsc.subcore_barrier()

    return pl.kernel(
        body,
        out_type=jax.ShapeDtypeStruct((nchunk * n, 32), jnp.float32),
        mesh=plsc.VectorSubcoreMesh(core_axis_name="c", subcore_axis_name="s",
                                    num_cores=NC, num_subcores=NS),
        scratch_types=[
            pltpu.VMEM_SHARED((acc_rows, 32), jnp.float32),
            pltpu.VMEM((blk, EB), jnp.int32),
            pltpu.VMEM((blk, EB), jnp.int32),
            pltpu.VMEM((blk, EB), jnp.int32),
            pltpu.VMEM((blk, EB, 32), jnp.float32),
            pltpu.VMEM((zrows, 32), jnp.float32),
        ],
    )


def _make_decode(n, d, lrows_pad, lblk):
    rows_w = lrows_pad // NW
    nblk = rows_w // lblk

    def body(z_hbm, ia_hbm, ib_hbm, za_hbm, zb_hbm, iabuf, ibbuf, rabuf, rbbuf):
        c = lax.axis_index("c")
        s = lax.axis_index("s")
        w = c * NS + s

        @pl.loop(0, nblk)
        def _(g):
            b0 = w * rows_w + g * lblk
            pltpu.sync_copy(ia_hbm.at[pl.ds(b0, lblk)], iabuf)
            pltpu.sync_copy(ib_hbm.at[pl.ds(b0, lblk)], ibbuf)
            for j in range(lblk):
                pltpu.sync_copy(z_hbm.at[iabuf.at[j]], rabuf.at[j])
                pltpu.sync_copy(z_hbm.at[ibbuf.at[j]], rbbuf.at[j])
            pltpu.sync_copy(rabuf, za_hbm.at[pl.ds(b0, lblk)])
            pltpu.sync_copy(rbbuf, zb_hbm.at[pl.ds(b0, lblk)])

    return pl.kernel(
        body,
        out_type=(jax.ShapeDtypeStruct((lrows_pad, EB, d), jnp.float32),
                  jax.ShapeDtypeStruct((lrows_pad, EB, d), jnp.float32)),
        mesh=plsc.VectorSubcoreMesh(core_axis_name="c", subcore_axis_name="s",
                                    num_cores=NC, num_subcores=NS),
        scratch_types=[
            pltpu.VMEM((lblk, EB), jnp.int32),
            pltpu.VMEM((lblk, EB), jnp.int32),
            pltpu.VMEM((lblk, EB, d), jnp.float32),
            pltpu.VMEM((lblk, EB, d), jnp.float32),
        ],
    )


def _make_mm1(n, din, d1, rb):
    def body(x_ref, w_ref, o_ref):
        o_ref[...] = jnp.dot(x_ref[...], w_ref[...],
                             preferred_element_type=jnp.float32)

    return pl.pallas_call(
        body,
        grid=(n // rb,),
        in_specs=[pl.BlockSpec((rb, din), lambda i: (i, 0)),
                  pl.BlockSpec((din, d1), lambda i: (0, 0))],
        out_specs=pl.BlockSpec((rb, d1), lambda i: (i, 0)),
        out_shape=jax.ShapeDtypeStruct((n, d1), jnp.float32),
    )


def _make_scale(n, d1, bt):
    nchunk = d1 // 32

    def body(deg_ref, h_ref, dinv_ref, hsc_ref):
        d = deg_ref[0] + deg_ref[1] + 1.0
        dinv = lax.rsqrt(d)
        dinv_ref[...] = dinv
        hs = h_ref[...] * dinv
        for cc in range(nchunk):
            hsc_ref[cc] = hs[:, cc * 32:(cc + 1) * 32]

    return pl.pallas_call(
        body,
        grid=(n // bt,),
        in_specs=[pl.BlockSpec((2, bt, 1), lambda i: (0, i, 0)),
                  pl.BlockSpec((bt, d1), lambda i: (i, 0))],
        out_specs=[pl.BlockSpec((bt, 1), lambda i: (i, 0)),
                   pl.BlockSpec((nchunk, bt, 32), lambda i: (0, i, 0))],
        out_shape=[jax.ShapeDtypeStruct((n, 1), jnp.float32),
                   jax.ShapeDtypeStruct((nchunk, n, 32), jnp.float32)],
    )


def _make_layer1_epilogue(n, d1, d2, bt):
    nc1, nc2 = d1 // 32, d2 // 32

    def body(acc_ref, hsc_ref, dinv_ref, b1_ref, w2_ref, hs2_ref):
        a = jnp.concatenate([acc_ref[cc] + hsc_ref[cc] for cc in range(nc1)],
                            axis=-1)
        dinv = dinv_ref[...]
        r1 = jnp.maximum(a * dinv + b1_ref[...], 0.0)
        h2 = jnp.dot(r1, w2_ref[...], preferred_element_type=jnp.float32)
        hs2 = h2 * dinv
        for cc in range(nc2):
            hs2_ref[cc] = hs2[:, cc * 32:(cc + 1) * 32]

    return pl.pallas_call(
        body,
        grid=(n // bt,),
        in_specs=[pl.BlockSpec((nc1, bt, 32), lambda i: (0, i, 0)),
                  pl.BlockSpec((nc1, bt, 32), lambda i: (0, i, 0)),
                  pl.BlockSpec((bt, 1), lambda i: (i, 0)),
                  pl.BlockSpec((1, d1), lambda i: (0, 0)),
                  pl.BlockSpec((d1, d2), lambda i: (0, 0))],
        out_specs=pl.BlockSpec((nc2, bt, 32), lambda i: (0, i, 0)),
        out_shape=jax.ShapeDtypeStruct((nc2, n, 32), jnp.float32),
    )


def _make_layer2_epilogue(n, d2, bt):
    nc2 = d2 // 32

    def body(acc_ref, hsc_ref, dinv_ref, b2_ref, z_ref):
        a = jnp.concatenate([acc_ref[cc] + hsc_ref[cc] for cc in range(nc2)],
                            axis=-1)
        z_ref[...] = a * dinv_ref[...] + b2_ref[...]

    return pl.pallas_call(
        body,
        grid=(n // bt,),
        in_specs=[pl.BlockSpec((nc2, bt, 32), lambda i: (0, i, 0)),
                  pl.BlockSpec((nc2, bt, 32), lambda i: (0, i, 0)),
                  pl.BlockSpec((bt, 1), lambda i: (i, 0)),
                  pl.BlockSpec((1, d2), lambda i: (0, 0))],
        out_specs=pl.BlockSpec((bt, d2), lambda i: (i, 0)),
        out_shape=jax.ShapeDtypeStruct((n, d2), jnp.float32),
    )


def _make_rowdot(m, d, bt):
    def body(a_ref, b_ref, o_ref):
        o_ref[...] = jnp.sum(a_ref[...] * b_ref[...], axis=1, keepdims=True)

    return pl.pallas_call(
        body,
        grid=(m // bt,),
        in_specs=[pl.BlockSpec((bt, d), lambda i: (i, 0)),
                  pl.BlockSpec((bt, d), lambda i: (i, 0))],
        out_specs=pl.BlockSpec((bt, 1), lambda i: (i, 0)),
        out_shape=jax.ShapeDtypeStruct((m, 1), jnp.float32),
    )


EROWS_PAD = _rup(_cdiv(E, EB), NS * 8)
LROWS_PAD = _rup(_cdiv(NLBL, EB), NW)

_deg = _make_deg(N, EROWS_PAD, db=4)
_agg4 = _make_agg(N, EROWS_PAD, nchunk=4, blk=8, zrows=125)
_agg2 = _make_agg(N, EROWS_PAD, nchunk=2, blk=8, zrows=125)
_decode = _make_decode(N, D2, LROWS_PAD, lblk=5)
_mm1 = _make_mm1(N, DIN, D1, rb=2500)
_scale = _make_scale(N, D1, bt=2000)
_epi1 = _make_layer1_epilogue(N, D1, D2, bt=2000)
_epi2 = _make_layer2_epilogue(N, D2, bt=2000)
_rowdot = _make_rowdot(LROWS_PAD * EB, D2, bt=2048)


def kernel(x, edge_index, edge_label_index, W1, b1, W2, b2):
    pad_e = EROWS_PAD * EB - E
    src2d = jnp.concatenate(
        [edge_index[0], jnp.zeros((pad_e,), jnp.int32)]).reshape(EROWS_PAD, EB)
    dst2d = jnp.concatenate(
        [edge_index[1], jnp.full((pad_e,), N, jnp.int32)]).reshape(EROWS_PAD, EB)

    deg2 = _deg(dst2d)
    h1 = _mm1(x, W1)
    dinv, hs1c = _scale(deg2.reshape(2, N, 1), h1)
    acc1 = _agg4(hs1c.reshape(4 * N, 32), src2d, dst2d)
    hs2c = _epi1(acc1.reshape(4, N, 32), hs1c, dinv,
                 b1.reshape(1, D1), W2)
    acc2 = _agg2(hs2c.reshape(2 * N, 32), src2d, dst2d)
    z = _epi2(acc2.reshape(2, N, 32), hs2c, dinv, b2.reshape(1, D2))

    pad_l = LROWS_PAD * EB - NLBL
    ia2d = jnp.concatenate(
        [edge_label_index[0], jnp.zeros((pad_l,), jnp.int32)]).reshape(LROWS_PAD, EB)
    ib2d = jnp.concatenate(
        [edge_label_index[1], jnp.zeros((pad_l,), jnp.int32)]).reshape(LROWS_PAD, EB)
    za, zb = _decode(z, ia2d, ib2d)
    dots = _rowdot(za.reshape(-1, D2), zb.reshape(-1, D2))
    return dots[:NLBL, 0]


# speedup vs baseline: 10.1330x; 10.1330x over previous
import jax
import jax.numpy as jnp
from jax import lax
from jax.experimental import pallas as pl
from jax.experimental.pallas import tpu as pltpu
from jax.experimental.pallas import tpu_sc as plsc

N = 50000
E = 800000
NLBL = 100000
DIN, D1, D2 = 500, 128, 64

NC, NS = 2, 16
NW = NC * NS

EB = 128


def _cdiv(a, b):
    return (a + b - 1) // b


def _rup(a, b):
    return _cdiv(a, b) * b


NPAD = _rup(N, 128)
EROWS_PAD = _rup(_cdiv(E, EB), NW * 8)
LROWS_PAD = _rup(_cdiv(NLBL, EB), NW * 8)

_TRNG = 3200
_TLAST = NPAD - (NS - 1) * _TRNG


def _tile_rows(s):
    start = s * _TRNG
    nz = jnp.where(s < NS - 1, _TRNG // 128, _TLAST // 128)
    return start, nz


def _make_deg(db):
    rows_w = EROWS_PAD // NW
    nblk = rows_w // db

    def body(dst_hbm, out0_hbm, out1_hbm, dacc, dbuf, ones_v, zbuf):
        c = lax.axis_index("c")
        s = lax.axis_index("s")
        w = c * NS + s

        @pl.loop(0, 8)
        def _(k):
            ones_v[pl.ds(k * 16, 16)] = jnp.ones((16,), jnp.float32)

        @pl.loop(0, _TRNG // 16)
        def _(k):
            zbuf[pl.ds(k * 16, 16)] = jnp.zeros((16,), jnp.float32)

        start, nz = _tile_rows(s)

        @pl.when(s < NS - 1)
        def _():
            pltpu.sync_copy(zbuf, dacc.at[pl.ds(start, _TRNG)])

        @pl.when(s == NS - 1)
        def _():
            pltpu.sync_copy(zbuf.at[pl.ds(0, _TLAST)], dacc.at[pl.ds(start, _TLAST)])

        plsc.subcore_barrier()

        @pl.loop(0, nblk)
        def _(g):
            pltpu.sync_copy(dst_hbm.at[pl.ds(w * rows_w + g * db, db)], dbuf)
            for j in range(db):
                pltpu.sync_copy(ones_v, dacc.at[dbuf.at[j]], add=True)

        plsc.subcore_barrier()

        for cc, out_hbm in ((0, out0_hbm), (1, out1_hbm)):
            @pl.when(c == cc)
            def _():
                @pl.when(s < NS - 1)
                def _():
                    pltpu.sync_copy(dacc.at[pl.ds(start, _TRNG)],
                                    out_hbm.at[pl.ds(start, _TRNG)])

                @pl.when(s == NS - 1)
                def _():
                    pltpu.sync_copy(dacc.at[pl.ds(start, _TLAST)],
                                    out_hbm.at[pl.ds(start, _TLAST)])

    return pl.kernel(
        body,
        out_type=(jax.ShapeDtypeStruct((NPAD,), jnp.float32),
                  jax.ShapeDtypeStruct((NPAD,), jnp.float32)),
        mesh=plsc.VectorSubcoreMesh(core_axis_name="c", subcore_axis_name="s",
                                    num_cores=NC, num_subcores=NS),
        compiler_params=pltpu.CompilerParams(use_tc_tiling_on_sc=False),
        scratch_types=[
            pltpu.VMEM_SHARED((NPAD,), jnp.float32),
            pltpu.VMEM((db, EB), jnp.int32),
            pltpu.VMEM((EB,), jnp.float32),
            pltpu.VMEM((_TRNG,), jnp.float32),
        ],
    )


def _make_agg(nchunk, blk):
    acc_rows = NPAD
    nr = nchunk // NC
    rows_t = EROWS_PAD // NS
    nblk = rows_t // blk

    def body(table, src_hbm, dst_hbm, out_hbm, acc, sbuf, dbuf, obuf, rbuf, zbuf):
        c = lax.axis_index("c")
        s = lax.axis_index("s")

        @pl.loop(0, 128)
        def _(i):
            zbuf[i, pl.ds(0, 16)] = jnp.zeros((16,), jnp.float32)
            zbuf[i, pl.ds(16, 16)] = jnp.zeros((16,), jnp.float32)

        start, nz = _tile_rows(s)

        for r in range(nr):
            chunk = c * nr + r
            off = chunk * NPAD

            @pl.loop(0, nz)
            def _(k):
                pltpu.sync_copy(zbuf, acc.at[pl.ds(start + k * 128, 128)])

            plsc.subcore_barrier()

            @pl.loop(0, nblk)
            def _(g):
                b0 = s * rows_t + g * blk
                pltpu.sync_copy(src_hbm.at[pl.ds(b0, blk)], sbuf)
                pltpu.sync_copy(dst_hbm.at[pl.ds(b0, blk)], dbuf)
                for j in range(blk):
                    for k in range(EB // 16):
                        obuf[j, pl.ds(k * 16, 16)] = sbuf[j, pl.ds(k * 16, 16)] + off
                for j in range(blk):
                    pltpu.sync_copy(table.at[obuf.at[j]], rbuf.at[j])
                    pltpu.sync_copy(rbuf.at[j], acc.at[dbuf.at[j]], add=True)

            plsc.subcore_barrier()

            @pl.loop(0, nz)
            def _(k):
                r0 = start + k * 128
                pltpu.sync_copy(acc.at[pl.ds(r0, 128)],
                                out_hbm.at[pl.ds(off + r0, 128)])

            plsc.subcore_barrier()

    return pl.kernel(
        body,
        out_type=jax.ShapeDtypeStruct((nchunk * NPAD, 32), jnp.float32),
        mesh=plsc.VectorSubcoreMesh(core_axis_name="c", subcore_axis_name="s",
                                    num_cores=NC, num_subcores=NS),
        compiler_params=pltpu.CompilerParams(use_tc_tiling_on_sc=False),
        scratch_types=[
            pltpu.VMEM_SHARED((acc_rows, 32), jnp.float32),
            pltpu.VMEM((blk, EB), jnp.int32),
            pltpu.VMEM((blk, EB), jnp.int32),
            pltpu.VMEM((blk, EB), jnp.int32),
            pltpu.VMEM((blk, EB, 32), jnp.float32),
            pltpu.VMEM((128, 32), jnp.float32),
        ],
    )


def _make_decode(d, lblk):
    rows_w = LROWS_PAD // NW
    nblk = rows_w // lblk

    def body(z_hbm, ia_hbm, ib_hbm, za_hbm, zb_hbm, ibuf, rbuf):
        c = lax.axis_index("c")
        s = lax.axis_index("s")
        w = c * NS + s

        @pl.loop(0, nblk)
        def _(g):
            b0 = w * rows_w + g * lblk
            for idx_hbm, o_hbm in ((ia_hbm, za_hbm), (ib_hbm, zb_hbm)):
                pltpu.sync_copy(idx_hbm.at[pl.ds(b0, lblk)], ibuf)
                for j in range(lblk):
                    pltpu.sync_copy(z_hbm.at[ibuf.at[j]], rbuf.at[j])
                pltpu.sync_copy(rbuf, o_hbm.at[pl.ds(b0, lblk)])

    return pl.kernel(
        body,
        out_type=(jax.ShapeDtypeStruct((LROWS_PAD, EB, d), jnp.float32),
                  jax.ShapeDtypeStruct((LROWS_PAD, EB, d), jnp.float32)),
        mesh=plsc.VectorSubcoreMesh(core_axis_name="c", subcore_axis_name="s",
                                    num_cores=NC, num_subcores=NS),
        compiler_params=pltpu.CompilerParams(use_tc_tiling_on_sc=False),
        scratch_types=[
            pltpu.VMEM((lblk, EB), jnp.int32),
            pltpu.VMEM((lblk, EB, d), jnp.float32),
        ],
    )


def _make_mm1(rb):
    def body(x_ref, w_ref, o_ref):
        o_ref[...] = jnp.dot(x_ref[...], w_ref[...],
                             preferred_element_type=jnp.float32)

    return pl.pallas_call(
        body,
        grid=(N // rb,),
        in_specs=[pl.BlockSpec((rb, DIN), lambda i: (i, 0)),
                  pl.BlockSpec((DIN, D1), lambda i: (0, 0))],
        out_specs=pl.BlockSpec((rb, D1), lambda i: (i, 0)),
        out_shape=jax.ShapeDtypeStruct((N, D1), jnp.float32),
    )


def _make_scale(bt):
    nchunk = D1 // 32

    def body(dga_ref, dgb_ref, h_ref, dinv_ref, hsc_ref):
        d = dga_ref[...] + dgb_ref[...] + 1.0
        dinv = lax.rsqrt(d)
        dinv_ref[...] = dinv
        hs = h_ref[...] * dinv
        for cc in range(nchunk):
            hsc_ref[cc] = hs[:, cc * 32:(cc + 1) * 32]

    return pl.pallas_call(
        body,
        grid=(N // bt,),
        in_specs=[pl.BlockSpec((bt, 1), lambda i: (i, 0)),
                  pl.BlockSpec((bt, 1), lambda i: (i, 0)),
                  pl.BlockSpec((bt, D1), lambda i: (i, 0))],
        out_specs=[pl.BlockSpec((bt, 1), lambda i: (i, 0)),
                   pl.BlockSpec((nchunk, bt, 32), lambda i: (0, i, 0))],
        out_shape=[jax.ShapeDtypeStruct((N, 1), jnp.float32),
                   jax.ShapeDtypeStruct((nchunk, NPAD, 32), jnp.float32)],
    )


def _make_layer1_epilogue(bt):
    nc1, nc2 = D1 // 32, D2 // 32

    def body(acc_ref, hsc_ref, dinv_ref, b1_ref, w2_ref, hs2_ref):
        a = jnp.concatenate([acc_ref[cc] + hsc_ref[cc] for cc in range(nc1)],
                            axis=-1)
        dinv = dinv_ref[...]
        r1 = jnp.maximum(a * dinv + b1_ref[...], 0.0)
        h2 = jnp.dot(r1, w2_ref[...], preferred_element_type=jnp.float32)
        hs2 = h2 * dinv
        for cc in range(nc2):
            hs2_ref[cc] = hs2[:, cc * 32:(cc + 1) * 32]

    return pl.pallas_call(
        body,
        grid=(N // bt,),
        in_specs=[pl.BlockSpec((nc1, bt, 32), lambda i: (0, i, 0)),
                  pl.BlockSpec((nc1, bt, 32), lambda i: (0, i, 0)),
                  pl.BlockSpec((bt, 1), lambda i: (i, 0)),
                  pl.BlockSpec((1, D1), lambda i: (0, 0)),
                  pl.BlockSpec((D1, D2), lambda i: (0, 0))],
        out_specs=pl.BlockSpec((nc2, bt, 32), lambda i: (0, i, 0)),
        out_shape=jax.ShapeDtypeStruct((nc2, NPAD, 32), jnp.float32),
    )


def _make_layer2_epilogue(bt):
    nc2 = D2 // 32

    def body(acc_ref, hsc_ref, dinv_ref, b2_ref, z_ref):
        a = jnp.concatenate([acc_ref[cc] + hsc_ref[cc] for cc in range(nc2)],
                            axis=-1)
        z_ref[...] = a * dinv_ref[...] + b2_ref[...]

    return pl.pallas_call(
        body,
        grid=(N // bt,),
        in_specs=[pl.BlockSpec((nc2, bt, 32), lambda i: (0, i, 0)),
                  pl.BlockSpec((nc2, bt, 32), lambda i: (0, i, 0)),
                  pl.BlockSpec((bt, 1), lambda i: (i, 0)),
                  pl.BlockSpec((1, D2), lambda i: (0, 0))],
        out_specs=pl.BlockSpec((bt, D2), lambda i: (i, 0)),
        out_shape=jax.ShapeDtypeStruct((N, D2), jnp.float32),
    )


def _make_rowdot(m, bt):
    def body(a_ref, b_ref, o_ref):
        o_ref[...] = jnp.sum(a_ref[...] * b_ref[...], axis=1, keepdims=True)

    return pl.pallas_call(
        body,
        grid=(m // bt,),
        in_specs=[pl.BlockSpec((bt, D2), lambda i: (i, 0)),
                  pl.BlockSpec((bt, D2), lambda i: (i, 0))],
        out_specs=pl.BlockSpec((bt, 1), lambda i: (i, 0)),
        out_shape=jax.ShapeDtypeStruct((m, 1), jnp.float32),
    )


_deg = _make_deg(db=8)
_agg4 = _make_agg(nchunk=4, blk=4)
_agg2 = _make_agg(nchunk=2, blk=4)
_decode = _make_decode(D2, lblk=8)
_mm1 = _make_mm1(rb=2000)
_scale = _make_scale(bt=2000)
_epi1 = _make_layer1_epilogue(bt=2000)
_epi2 = _make_layer2_epilogue(bt=2000)
_rowdot = _make_rowdot(LROWS_PAD * EB, bt=2048)


def kernel(x, edge_index, edge_label_index, W1, b1, W2, b2):
    pad_e = EROWS_PAD * EB - E
    pad_src = (jnp.arange(pad_e, dtype=jnp.int32) * 61) % 4096
    pad_dst = N + (jnp.arange(pad_e, dtype=jnp.int32) % (NPAD - N))
    src2d = jnp.concatenate([edge_index[0], pad_src]).reshape(EROWS_PAD, EB)
    dst2d = jnp.concatenate([edge_index[1], pad_dst]).reshape(EROWS_PAD, EB)

    dga, dgb = _deg(dst2d)
    h1 = _mm1(x, W1)
    dinv, hs1c = _scale(dga[:N].reshape(N, 1), dgb[:N].reshape(N, 1), h1)
    acc1 = _agg4(hs1c.reshape(4 * NPAD, 32), src2d, dst2d)
    hs2c = _epi1(acc1.reshape(4, NPAD, 32), hs1c, dinv,
                 b1.reshape(1, D1), W2)
    acc2 = _agg2(hs2c.reshape(2 * NPAD, 32), src2d, dst2d)
    z = _epi2(acc2.reshape(2, NPAD, 32), hs2c, dinv, b2.reshape(1, D2))

    pad_l = LROWS_PAD * EB - NLBL
    pad_i = (jnp.arange(pad_l, dtype=jnp.int32) * 61) % 4096
    ia2d = jnp.concatenate([edge_label_index[0], pad_i]).reshape(LROWS_PAD, EB)
    ib2d = jnp.concatenate([edge_label_index[1], pad_i]).reshape(LROWS_PAD, EB)
    za, zb = _decode(z, ia2d, ib2d)
    dots = _rowdot(za.reshape(-1, D2), zb.reshape(-1, D2))
    return dots[:NLBL, 0]


# speedup vs baseline: 16.0644x; 1.5854x over previous
import jax
import jax.numpy as jnp
from jax import lax
from jax.experimental import pallas as pl
from jax.experimental.pallas import tpu as pltpu
from jax.experimental.pallas import tpu_sc as plsc

N = 50000
E = 800000
NLBL = 100000
DIN, D1, D2 = 500, 128, 64

NC, NS = 2, 16
NW = NC * NS

EB = 128


def _cdiv(a, b):
    return (a + b - 1) // b


def _rup(a, b):
    return _cdiv(a, b) * b


NPAD = _rup(N, 128)
EROWS_PAD = _rup(_cdiv(E, EB), NW * 8)
LROWS_PAD = _rup(_cdiv(NLBL, EB), NW * 8)

_TRNG = 3200
_TLAST = NPAD - (NS - 1) * _TRNG


def _tile_rows(s):
    start = s * _TRNG
    nz = jnp.where(s < NS - 1, _TRNG // 128, _TLAST // 128)
    return start, nz


def _make_deg(db):
    rows_w = EROWS_PAD // NW
    nblk = rows_w // db

    def body(dst_hbm, out0_hbm, out1_hbm, dacc, dbuf, ones_v, zbuf):
        c = lax.axis_index("c")
        s = lax.axis_index("s")
        w = c * NS + s

        @pl.loop(0, 8)
        def _(k):
            ones_v[pl.ds(k * 16, 16)] = jnp.ones((16,), jnp.float32)

        @pl.loop(0, _TRNG // 16)
        def _(k):
            zbuf[pl.ds(k * 16, 16)] = jnp.zeros((16,), jnp.float32)

        start, nz = _tile_rows(s)

        @pl.when(s < NS - 1)
        def _():
            pltpu.sync_copy(zbuf, dacc.at[pl.ds(start, _TRNG)])

        @pl.when(s == NS - 1)
        def _():
            pltpu.sync_copy(zbuf.at[pl.ds(0, _TLAST)], dacc.at[pl.ds(start, _TLAST)])

        plsc.subcore_barrier()

        @pl.loop(0, nblk)
        def _(g):
            pltpu.sync_copy(dst_hbm.at[pl.ds(w * rows_w + g * db, db)], dbuf)
            for j in range(db):
                pltpu.sync_copy(ones_v, dacc.at[dbuf.at[j]], add=True)

        plsc.subcore_barrier()

        for cc, out_hbm in ((0, out0_hbm), (1, out1_hbm)):
            @pl.when(c == cc)
            def _():
                @pl.when(s < NS - 1)
                def _():
                    pltpu.sync_copy(dacc.at[pl.ds(start, _TRNG)],
                                    out_hbm.at[pl.ds(start, _TRNG)])

                @pl.when(s == NS - 1)
                def _():
                    pltpu.sync_copy(dacc.at[pl.ds(start, _TLAST)],
                                    out_hbm.at[pl.ds(start, _TLAST)])

    return pl.kernel(
        body,
        out_type=(jax.ShapeDtypeStruct((NPAD,), jnp.float32),
                  jax.ShapeDtypeStruct((NPAD,), jnp.float32)),
        mesh=plsc.VectorSubcoreMesh(core_axis_name="c", subcore_axis_name="s",
                                    num_cores=NC, num_subcores=NS),
        compiler_params=pltpu.CompilerParams(use_tc_tiling_on_sc=False),
        scratch_types=[
            pltpu.VMEM_SHARED((NPAD,), jnp.float32),
            pltpu.VMEM((db, EB), jnp.int32),
            pltpu.VMEM((EB,), jnp.float32),
            pltpu.VMEM((_TRNG,), jnp.float32),
        ],
    )


def _make_agg(nchunk, blk):
    acc_rows = NPAD
    nr = nchunk // NC
    rows_t = EROWS_PAD // NS
    K = blk
    G = rows_t // K
    PAIRS = G // 2

    def body(table, src_hbm, dst_hbm, out_hbm, acc, sring, dring, oring, rbuf,
             zbuf, isem, gsem, ssem):
        c = lax.axis_index("c")
        s = lax.axis_index("s")

        @pl.loop(0, 128)
        def _(i):
            zbuf[i, pl.ds(0, 16)] = jnp.zeros((16,), jnp.float32)
            zbuf[i, pl.ds(16, 16)] = jnp.zeros((16,), jnp.float32)

        start, nz = _tile_rows(s)
        base = s * rows_t

        def issue_idx(g, h):
            pltpu.make_async_copy(src_hbm.at[pl.ds(base + g * K, K)],
                                  sring.at[h], isem).start()
            pltpu.make_async_copy(dst_hbm.at[pl.ds(base + g * K, K)],
                                  dring.at[h], isem).start()

        def wait_idx(h):
            for _ in range(2):
                pltpu.make_async_copy(src_hbm.at[pl.ds(base, K)],
                                      sring.at[h], isem).wait()

        def wait_row(sem):
            pltpu.make_async_copy(table.at[pl.ds(0, 128)], rbuf.at[0, 0],
                                  sem).wait()

        for r in range(nr):
            chunk = c * nr + r
            off = chunk * NPAD

            @pl.loop(0, nz)
            def _(k):
                pltpu.sync_copy(zbuf, acc.at[pl.ds(start + k * 128, 128)])

            plsc.subcore_barrier()

            issue_idx(0, 0)

            @pl.loop(0, PAIRS)
            def _(p):
                for h in range(2):
                    g = 2 * p + h
                    wait_idx(h)
                    for j in range(K):
                        for k in range(EB // 16):
                            oring[h, j, pl.ds(k * 16, 16)] = (
                                sring[h, j, pl.ds(k * 16, 16)] + off)
                    for j in range(K):
                        pltpu.make_async_copy(table.at[oring.at[h, j]],
                                              rbuf.at[h, j], gsem).start()
                    if h == 0:
                        @pl.when(p > 0)
                        def _():
                            for j in range(K):
                                wait_row(ssem)

                        issue_idx(g + 1, 1 - h)
                    else:
                        for j in range(K):
                            wait_row(ssem)

                        @pl.when(p < PAIRS - 1)
                        def _():
                            issue_idx(g + 1, 1 - h)
                    for j in range(K):
                        wait_row(gsem)
                    for j in range(K):
                        pltpu.make_async_copy(
                            rbuf.at[h, j], acc.at[dring.at[h, j]],
                            ssem).start(add=True)

            for j in range(K):
                wait_row(ssem)

            plsc.subcore_barrier()

            @pl.loop(0, nz)
            def _(k):
                r0 = start + k * 128
                pltpu.sync_copy(acc.at[pl.ds(r0, 128)],
                                out_hbm.at[pl.ds(off + r0, 128)])

            plsc.subcore_barrier()

    return pl.kernel(
        body,
        out_type=jax.ShapeDtypeStruct((nchunk * NPAD, 32), jnp.float32),
        mesh=plsc.VectorSubcoreMesh(core_axis_name="c", subcore_axis_name="s",
                                    num_cores=NC, num_subcores=NS),
        compiler_params=pltpu.CompilerParams(use_tc_tiling_on_sc=False),
        scratch_types=[
            pltpu.VMEM_SHARED((acc_rows, 32), jnp.float32),
            pltpu.VMEM((2, blk, EB), jnp.int32),
            pltpu.VMEM((2, blk, EB), jnp.int32),
            pltpu.VMEM((2, blk, EB), jnp.int32),
            pltpu.VMEM((2, blk, EB, 32), jnp.float32),
            pltpu.VMEM((128, 32), jnp.float32),
            pltpu.SemaphoreType.DMA,
            pltpu.SemaphoreType.DMA,
            pltpu.SemaphoreType.DMA,
        ],
    )


def _make_decode(d, lblk):
    del lblk
    rows_w = LROWS_PAD // NW

    def body(z_hbm, ia_hbm, ib_hbm, za_hbm, zb_hbm, iabuf, ibbuf, rbuf,
             gsem, osem):
        c = lax.axis_index("c")
        s = lax.axis_index("s")
        w = c * NS + s
        b0 = w * rows_w

        pltpu.sync_copy(ia_hbm.at[pl.ds(b0, rows_w)], iabuf)
        pltpu.sync_copy(ib_hbm.at[pl.ds(b0, rows_w)], ibbuf)

        def wait_row(sem):
            pltpu.make_async_copy(z_hbm.at[pl.ds(0, EB)], rbuf.at[0, 0],
                                  sem).wait()

        for g in range(rows_w):
            h = g % 2
            if g >= 2:
                wait_row(osem)
                wait_row(osem)
            pltpu.make_async_copy(z_hbm.at[iabuf.at[g]], rbuf.at[h, 0],
                                  gsem).start()
            pltpu.make_async_copy(z_hbm.at[ibbuf.at[g]], rbuf.at[h, 1],
                                  gsem).start()
            if g >= 1:
                wait_row(gsem)
                wait_row(gsem)
                pltpu.make_async_copy(rbuf.at[1 - h, 0],
                                      za_hbm.at[b0 + g - 1], osem).start()
                pltpu.make_async_copy(rbuf.at[1 - h, 1],
                                      zb_hbm.at[b0 + g - 1], osem).start()
        wait_row(gsem)
        wait_row(gsem)
        h = (rows_w - 1) % 2
        pltpu.make_async_copy(rbuf.at[h, 0], za_hbm.at[b0 + rows_w - 1],
                              osem).start()
        pltpu.make_async_copy(rbuf.at[h, 1], zb_hbm.at[b0 + rows_w - 1],
                              osem).start()
        for _ in range(4):
            wait_row(osem)

    return pl.kernel(
        body,
        out_type=(jax.ShapeDtypeStruct((LROWS_PAD, EB, d), jnp.float32),
                  jax.ShapeDtypeStruct((LROWS_PAD, EB, d), jnp.float32)),
        mesh=plsc.VectorSubcoreMesh(core_axis_name="c", subcore_axis_name="s",
                                    num_cores=NC, num_subcores=NS),
        compiler_params=pltpu.CompilerParams(use_tc_tiling_on_sc=False),
        scratch_types=[
            pltpu.VMEM((LROWS_PAD // NW, EB), jnp.int32),
            pltpu.VMEM((LROWS_PAD // NW, EB), jnp.int32),
            pltpu.VMEM((2, 2, EB, d), jnp.float32),
            pltpu.SemaphoreType.DMA,
            pltpu.SemaphoreType.DMA,
        ],
    )


def _make_mm1(rb):
    def body(x_ref, w_ref, o_ref):
        o_ref[...] = jnp.dot(x_ref[...], w_ref[...],
                             preferred_element_type=jnp.float32)

    return pl.pallas_call(
        body,
        grid=(N // rb,),
        in_specs=[pl.BlockSpec((rb, DIN), lambda i: (i, 0)),
                  pl.BlockSpec((DIN, D1), lambda i: (0, 0))],
        out_specs=pl.BlockSpec((rb, D1), lambda i: (i, 0)),
        out_shape=jax.ShapeDtypeStruct((N, D1), jnp.float32),
    )


def _make_scale(bt):
    nchunk = D1 // 32

    def body(dga_ref, dgb_ref, h_ref, dinv_ref, hsc_ref):
        d = dga_ref[...] + dgb_ref[...] + 1.0
        dinv = lax.rsqrt(d)
        dinv_ref[...] = dinv
        hs = h_ref[...] * dinv
        for cc in range(nchunk):
            hsc_ref[cc] = hs[:, cc * 32:(cc + 1) * 32]

    return pl.pallas_call(
        body,
        grid=(N // bt,),
        in_specs=[pl.BlockSpec((bt, 1), lambda i: (i, 0)),
                  pl.BlockSpec((bt, 1), lambda i: (i, 0)),
                  pl.BlockSpec((bt, D1), lambda i: (i, 0))],
        out_specs=[pl.BlockSpec((bt, 1), lambda i: (i, 0)),
                   pl.BlockSpec((nchunk, bt, 32), lambda i: (0, i, 0))],
        out_shape=[jax.ShapeDtypeStruct((N, 1), jnp.float32),
                   jax.ShapeDtypeStruct((nchunk, NPAD, 32), jnp.float32)],
    )


def _make_layer1_epilogue(bt):
    nc1, nc2 = D1 // 32, D2 // 32

    def body(acc_ref, hsc_ref, dinv_ref, b1_ref, w2_ref, hs2_ref):
        a = jnp.concatenate([acc_ref[cc] + hsc_ref[cc] for cc in range(nc1)],
                            axis=-1)
        dinv = dinv_ref[...]
        r1 = jnp.maximum(a * dinv + b1_ref[...], 0.0)
        h2 = jnp.dot(r1, w2_ref[...], preferred_element_type=jnp.float32)
        hs2 = h2 * dinv
        for cc in range(nc2):
            hs2_ref[cc] = hs2[:, cc * 32:(cc + 1) * 32]

    return pl.pallas_call(
        body,
        grid=(N // bt,),
        in_specs=[pl.BlockSpec((nc1, bt, 32), lambda i: (0, i, 0)),
                  pl.BlockSpec((nc1, bt, 32), lambda i: (0, i, 0)),
                  pl.BlockSpec((bt, 1), lambda i: (i, 0)),
                  pl.BlockSpec((1, D1), lambda i: (0, 0)),
                  pl.BlockSpec((D1, D2), lambda i: (0, 0))],
        out_specs=pl.BlockSpec((nc2, bt, 32), lambda i: (0, i, 0)),
        out_shape=jax.ShapeDtypeStruct((nc2, NPAD, 32), jnp.float32),
    )


def _make_layer2_epilogue(bt):
    nc2 = D2 // 32

    def body(acc_ref, hsc_ref, dinv_ref, b2_ref, z_ref):
        a = jnp.concatenate([acc_ref[cc] + hsc_ref[cc] for cc in range(nc2)],
                            axis=-1)
        z_ref[...] = a * dinv_ref[...] + b2_ref[...]

    return pl.pallas_call(
        body,
        grid=(N // bt,),
        in_specs=[pl.BlockSpec((nc2, bt, 32), lambda i: (0, i, 0)),
                  pl.BlockSpec((nc2, bt, 32), lambda i: (0, i, 0)),
                  pl.BlockSpec((bt, 1), lambda i: (i, 0)),
                  pl.BlockSpec((1, D2), lambda i: (0, 0))],
        out_specs=pl.BlockSpec((bt, D2), lambda i: (i, 0)),
        out_shape=jax.ShapeDtypeStruct((N, D2), jnp.float32),
    )


def _make_rowdot(m, bt):
    def body(a_ref, b_ref, o_ref):
        o_ref[...] = jnp.sum(a_ref[...] * b_ref[...], axis=1, keepdims=True)

    return pl.pallas_call(
        body,
        grid=(m // bt,),
        in_specs=[pl.BlockSpec((bt, D2), lambda i: (i, 0)),
                  pl.BlockSpec((bt, D2), lambda i: (i, 0))],
        out_specs=pl.BlockSpec((bt, 1), lambda i: (i, 0)),
        out_shape=jax.ShapeDtypeStruct((m, 1), jnp.float32),
    )


_deg = _make_deg(db=8)
_agg4 = _make_agg(nchunk=4, blk=2)
_agg2 = _make_agg(nchunk=2, blk=2)
_decode = _make_decode(D2, lblk=8)
_mm1 = _make_mm1(rb=2000)
_scale = _make_scale(bt=2000)
_epi1 = _make_layer1_epilogue(bt=2000)
_epi2 = _make_layer2_epilogue(bt=2000)
_rowdot = _make_rowdot(LROWS_PAD * EB, bt=2048)


def kernel(x, edge_index, edge_label_index, W1, b1, W2, b2):
    pad_e = EROWS_PAD * EB - E
    pad_src = (jnp.arange(pad_e, dtype=jnp.int32) * 61) % 4096
    pad_dst = N + (jnp.arange(pad_e, dtype=jnp.int32) % (NPAD - N))
    src2d = jnp.concatenate([edge_index[0], pad_src]).reshape(EROWS_PAD, EB)
    dst2d = jnp.concatenate([edge_index[1], pad_dst]).reshape(EROWS_PAD, EB)

    dga, dgb = _deg(dst2d)
    h1 = _mm1(x, W1)
    dinv, hs1c = _scale(dga[:N].reshape(N, 1), dgb[:N].reshape(N, 1), h1)
    acc1 = _agg4(hs1c.reshape(4 * NPAD, 32), src2d, dst2d)
    hs2c = _epi1(acc1.reshape(4, NPAD, 32), hs1c, dinv,
                 b1.reshape(1, D1), W2)
    acc2 = _agg2(hs2c.reshape(2 * NPAD, 32), src2d, dst2d)
    z = _epi2(acc2.reshape(2, NPAD, 32), hs2c, dinv, b2.reshape(1, D2))

    pad_l = LROWS_PAD * EB - NLBL
    pad_i = (jnp.arange(pad_l, dtype=jnp.int32) * 61) % 4096
    ia2d = jnp.concatenate([edge_label_index[0], pad_i]).reshape(LROWS_PAD, EB)
    ib2d = jnp.concatenate([edge_label_index[1], pad_i]).reshape(LROWS_PAD, EB)
    za, zb = _decode(z, ia2d, ib2d)
    dots = _rowdot(za.reshape(-1, D2), zb.reshape(-1, D2))
    return dots[:NLBL, 0]


# speedup vs baseline: 19.2162x; 1.1962x over previous
import jax
import jax.numpy as jnp
from jax import lax
from jax.experimental import pallas as pl
from jax.experimental.pallas import tpu as pltpu
from jax.experimental.pallas import tpu_sc as plsc

N = 50000
E = 800000
NLBL = 100000
DIN, D1, D2 = 500, 128, 64

NC, NS = 2, 16
NW = NC * NS

EB = 128


def _cdiv(a, b):
    return (a + b - 1) // b


def _rup(a, b):
    return _cdiv(a, b) * b


NPAD = _rup(N, 128)
EROWS_PAD = _rup(_cdiv(E, EB), NW * 8)
LROWS_PAD = _rup(_cdiv(NLBL, EB), NW * 8)

_TRNG = 3200
_TLAST = NPAD - (NS - 1) * _TRNG


def _tile_rows(s):
    start = s * _TRNG
    nz = jnp.where(s < NS - 1, _TRNG // 128, _TLAST // 128)
    return start, nz


def _make_deg(db):
    rows_w = EROWS_PAD // NW
    nblk = rows_w // db

    def body(dst_hbm, out0_hbm, out1_hbm, dacc, dbuf, ones_v, zbuf):
        c = lax.axis_index("c")
        s = lax.axis_index("s")
        w = c * NS + s

        @pl.loop(0, 8)
        def _(k):
            ones_v[pl.ds(k * 16, 16)] = jnp.ones((16,), jnp.float32)

        @pl.loop(0, _TRNG // 16)
        def _(k):
            zbuf[pl.ds(k * 16, 16)] = jnp.zeros((16,), jnp.float32)

        start, nz = _tile_rows(s)

        @pl.when(s < NS - 1)
        def _():
            pltpu.sync_copy(zbuf, dacc.at[pl.ds(start, _TRNG)])

        @pl.when(s == NS - 1)
        def _():
            pltpu.sync_copy(zbuf.at[pl.ds(0, _TLAST)], dacc.at[pl.ds(start, _TLAST)])

        plsc.subcore_barrier()

        @pl.loop(0, nblk)
        def _(g):
            pltpu.sync_copy(dst_hbm.at[pl.ds(w * rows_w + g * db, db)], dbuf)
            for j in range(db):
                pltpu.sync_copy(ones_v, dacc.at[dbuf.at[j]], add=True)

        plsc.subcore_barrier()

        for cc, out_hbm in ((0, out0_hbm), (1, out1_hbm)):
            @pl.when(c == cc)
            def _():
                @pl.when(s < NS - 1)
                def _():
                    pltpu.sync_copy(dacc.at[pl.ds(start, _TRNG)],
                                    out_hbm.at[pl.ds(start, _TRNG)])

                @pl.when(s == NS - 1)
                def _():
                    pltpu.sync_copy(dacc.at[pl.ds(start, _TLAST)],
                                    out_hbm.at[pl.ds(start, _TLAST)])

    return pl.kernel(
        body,
        out_type=(jax.ShapeDtypeStruct((NPAD,), jnp.float32),
                  jax.ShapeDtypeStruct((NPAD,), jnp.float32)),
        mesh=plsc.VectorSubcoreMesh(core_axis_name="c", subcore_axis_name="s",
                                    num_cores=NC, num_subcores=NS),
        compiler_params=pltpu.CompilerParams(use_tc_tiling_on_sc=False),
        scratch_types=[
            pltpu.VMEM_SHARED((NPAD,), jnp.float32),
            pltpu.VMEM((db, EB), jnp.int32),
            pltpu.VMEM((EB,), jnp.float32),
            pltpu.VMEM((_TRNG,), jnp.float32),
        ],
    )


def _make_agg(nchunk):
    nr = nchunk // NC
    rows_t = EROWS_PAD // NS
    UNROLL = 8
    P = rows_t // UNROLL

    def body(table, src_hbm, dst_hbm, out_hbm, acc, sring, dring, oring, rbuf,
             zbuf, isem, gsem, ssem):
        c = lax.axis_index("c")
        s = lax.axis_index("s")

        @pl.loop(0, 64)
        def _(i):
            zbuf[i, pl.ds(0, 16)] = jnp.zeros((16,), jnp.float32)
            zbuf[i, pl.ds(16, 16)] = jnp.zeros((16,), jnp.float32)

        start, nz = _tile_rows(s)
        base = s * rows_t

        def issue_idx(row, v):
            pltpu.make_async_copy(src_hbm.at[base + row], sring.at[v],
                                  isem).start()
            pltpu.make_async_copy(dst_hbm.at[base + row], dring.at[v],
                                  isem).start()

        def wait_idx(v):
            for _ in range(2):
                pltpu.make_async_copy(src_hbm.at[base], sring.at[v],
                                      isem).wait()

        def wait_row(sem):
            pltpu.make_async_copy(table.at[pl.ds(0, 128)], rbuf.at[0],
                                  sem).wait()

        for r in range(nr):
            chunk = c * nr + r
            off = chunk * NPAD

            @pl.loop(0, nz * 2)
            def _(k):
                pltpu.sync_copy(zbuf, acc.at[pl.ds(start + k * 64, 64)])

            plsc.subcore_barrier()

            for v in range(4):
                issue_idx(v, v)

            @pl.loop(0, P)
            def _(p):
                for u in range(UNROLL):
                    g = UNROLL * p + u
                    s4, s8 = u % 4, u

                    def steady(u=u, s4=s4, s8=s8, g=g):
                        wait_row(ssem)

                    if u < 4:
                        pl.when(p > 0)(steady)
                    else:
                        steady()
                    wait_idx(s8)
                    for k in range(EB // 16):
                        oring[s4, pl.ds(k * 16, 16)] = (
                            sring[s8, pl.ds(k * 16, 16)] + off)
                    pltpu.make_async_copy(table.at[oring.at[s4]],
                                          rbuf.at[s4], gsem).start()

                    def prefetch(g=g, v=(u + 4) % 8):
                        issue_idx(g + 4, v)

                    if u < 4:
                        prefetch()
                    else:
                        pl.when(p < P - 1)(prefetch)

                    def drain_gather(s4m2=(u - 2) % 4, s8m2=(u - 2) % 8):
                        wait_row(gsem)
                        pltpu.make_async_copy(
                            rbuf.at[s4m2], acc.at[dring.at[s8m2]],
                            ssem).start(add=True)

                    if u < 2:
                        pl.when(p > 0)(drain_gather)
                    else:
                        drain_gather()

            for u in (UNROLL - 2, UNROLL - 1):
                wait_row(gsem)
                pltpu.make_async_copy(rbuf.at[u % 4], acc.at[dring.at[u]],
                                      ssem).start(add=True)
            for _ in range(4):
                wait_row(ssem)

            plsc.subcore_barrier()

            @pl.loop(0, nz)
            def _(k):
                r0 = start + k * 128
                pltpu.sync_copy(acc.at[pl.ds(r0, 128)],
                                out_hbm.at[pl.ds(off + r0, 128)])

            plsc.subcore_barrier()

    return pl.kernel(
        body,
        out_type=jax.ShapeDtypeStruct((nchunk * NPAD, 32), jnp.float32),
        mesh=plsc.VectorSubcoreMesh(core_axis_name="c", subcore_axis_name="s",
                                    num_cores=NC, num_subcores=NS),
        compiler_params=pltpu.CompilerParams(use_tc_tiling_on_sc=False),
        scratch_types=[
            pltpu.VMEM_SHARED((NPAD, 32), jnp.float32),
            pltpu.VMEM((8, EB), jnp.int32),
            pltpu.VMEM((8, EB), jnp.int32),
            pltpu.VMEM((4, EB), jnp.int32),
            pltpu.VMEM((4, EB, 32), jnp.float32),
            pltpu.VMEM((64, 32), jnp.float32),
            pltpu.SemaphoreType.DMA,
            pltpu.SemaphoreType.DMA,
            pltpu.SemaphoreType.DMA,
        ],
    )


def _make_decode(d, lblk):
    del lblk
    rows_w = LROWS_PAD // NW

    def body(z_hbm, ia_hbm, ib_hbm, za_hbm, zb_hbm, iabuf, ibbuf, rbuf,
             gsem, osem):
        c = lax.axis_index("c")
        s = lax.axis_index("s")
        w = c * NS + s
        b0 = w * rows_w

        pltpu.sync_copy(ia_hbm.at[pl.ds(b0, rows_w)], iabuf)
        pltpu.sync_copy(ib_hbm.at[pl.ds(b0, rows_w)], ibbuf)

        def wait_row(sem):
            pltpu.make_async_copy(z_hbm.at[pl.ds(0, EB)], rbuf.at[0, 0],
                                  sem).wait()

        for g in range(rows_w):
            h = g % 2
            if g >= 2:
                wait_row(osem)
                wait_row(osem)
            pltpu.make_async_copy(z_hbm.at[iabuf.at[g]], rbuf.at[h, 0],
                                  gsem).start()
            pltpu.make_async_copy(z_hbm.at[ibbuf.at[g]], rbuf.at[h, 1],
                                  gsem).start()
            if g >= 1:
                wait_row(gsem)
                wait_row(gsem)
                pltpu.make_async_copy(rbuf.at[1 - h, 0],
                                      za_hbm.at[b0 + g - 1], osem).start()
                pltpu.make_async_copy(rbuf.at[1 - h, 1],
                                      zb_hbm.at[b0 + g - 1], osem).start()
        wait_row(gsem)
        wait_row(gsem)
        h = (rows_w - 1) % 2
        pltpu.make_async_copy(rbuf.at[h, 0], za_hbm.at[b0 + rows_w - 1],
                              osem).start()
        pltpu.make_async_copy(rbuf.at[h, 1], zb_hbm.at[b0 + rows_w - 1],
                              osem).start()
        for _ in range(4):
            wait_row(osem)

    return pl.kernel(
        body,
        out_type=(jax.ShapeDtypeStruct((LROWS_PAD, EB, d), jnp.float32),
                  jax.ShapeDtypeStruct((LROWS_PAD, EB, d), jnp.float32)),
        mesh=plsc.VectorSubcoreMesh(core_axis_name="c", subcore_axis_name="s",
                                    num_cores=NC, num_subcores=NS),
        compiler_params=pltpu.CompilerParams(use_tc_tiling_on_sc=False),
        scratch_types=[
            pltpu.VMEM((LROWS_PAD // NW, EB), jnp.int32),
            pltpu.VMEM((LROWS_PAD // NW, EB), jnp.int32),
            pltpu.VMEM((2, 2, EB, d), jnp.float32),
            pltpu.SemaphoreType.DMA,
            pltpu.SemaphoreType.DMA,
        ],
    )


def _make_mm1(rb):
    nchunk = D1 // 32

    def body(x_ref, w_ref, dga_ref, dgb_ref, dinv_ref, hsc_ref):
        h = jnp.dot(x_ref[...], w_ref[...], preferred_element_type=jnp.float32)
        d = dga_ref[...] + dgb_ref[...] + 1.0
        dinv = lax.rsqrt(d)
        dinv_ref[...] = dinv
        hs = h * dinv
        for cc in range(nchunk):
            hsc_ref[cc] = hs[:, cc * 32:(cc + 1) * 32]

    return pl.pallas_call(
        body,
        grid=(N // rb,),
        in_specs=[pl.BlockSpec((rb, DIN), lambda i: (i, 0)),
                  pl.BlockSpec((DIN, D1), lambda i: (0, 0)),
                  pl.BlockSpec((rb, 1), lambda i: (i, 0)),
                  pl.BlockSpec((rb, 1), lambda i: (i, 0))],
        out_specs=[pl.BlockSpec((rb, 1), lambda i: (i, 0)),
                   pl.BlockSpec((nchunk, rb, 32), lambda i: (0, i, 0))],
        out_shape=[jax.ShapeDtypeStruct((N, 1), jnp.float32),
                   jax.ShapeDtypeStruct((nchunk, NPAD, 32), jnp.float32)],
    )


def _make_layer1_epilogue(bt):
    nc1, nc2 = D1 // 32, D2 // 32

    def body(acc_ref, hsc_ref, dinv_ref, b1_ref, w2_ref, hs2_ref):
        a = jnp.concatenate([acc_ref[cc] + hsc_ref[cc] for cc in range(nc1)],
                            axis=-1)
        dinv = dinv_ref[...]
        r1 = jnp.maximum(a * dinv + b1_ref[...], 0.0)
        h2 = jnp.dot(r1, w2_ref[...], preferred_element_type=jnp.float32)
        hs2 = h2 * dinv
        for cc in range(nc2):
            hs2_ref[cc] = hs2[:, cc * 32:(cc + 1) * 32]

    return pl.pallas_call(
        body,
        grid=(N // bt,),
        in_specs=[pl.BlockSpec((nc1, bt, 32), lambda i: (0, i, 0)),
                  pl.BlockSpec((nc1, bt, 32), lambda i: (0, i, 0)),
                  pl.BlockSpec((bt, 1), lambda i: (i, 0)),
                  pl.BlockSpec((1, D1), lambda i: (0, 0)),
                  pl.BlockSpec((D1, D2), lambda i: (0, 0))],
        out_specs=pl.BlockSpec((nc2, bt, 32), lambda i: (0, i, 0)),
        out_shape=jax.ShapeDtypeStruct((nc2, NPAD, 32), jnp.float32),
    )


def _make_layer2_epilogue(bt):
    nc2 = D2 // 32

    def body(acc_ref, hsc_ref, dinv_ref, b2_ref, z_ref):
        a = jnp.concatenate([acc_ref[cc] + hsc_ref[cc] for cc in range(nc2)],
                            axis=-1)
        z_ref[...] = a * dinv_ref[...] + b2_ref[...]

    return pl.pallas_call(
        body,
        grid=(N // bt,),
        in_specs=[pl.BlockSpec((nc2, bt, 32), lambda i: (0, i, 0)),
                  pl.BlockSpec((nc2, bt, 32), lambda i: (0, i, 0)),
                  pl.BlockSpec((bt, 1), lambda i: (i, 0)),
                  pl.BlockSpec((1, D2), lambda i: (0, 0))],
        out_specs=pl.BlockSpec((bt, D2), lambda i: (i, 0)),
        out_shape=jax.ShapeDtypeStruct((N, D2), jnp.float32),
    )


def _make_rowdot(m, bt):
    def body(a_ref, b_ref, o_ref):
        o_ref[...] = jnp.sum(a_ref[...] * b_ref[...], axis=1, keepdims=True)

    return pl.pallas_call(
        body,
        grid=(m // bt,),
        in_specs=[pl.BlockSpec((bt, D2), lambda i: (i, 0)),
                  pl.BlockSpec((bt, D2), lambda i: (i, 0))],
        out_specs=pl.BlockSpec((bt, 1), lambda i: (i, 0)),
        out_shape=jax.ShapeDtypeStruct((m, 1), jnp.float32),
    )


_deg = _make_deg(db=8)
_agg4 = _make_agg(nchunk=4)
_agg2 = _make_agg(nchunk=2)
_decode = _make_decode(D2, lblk=8)
_mm1 = _make_mm1(rb=2000)
_epi1 = _make_layer1_epilogue(bt=2000)
_epi2 = _make_layer2_epilogue(bt=2000)
_rowdot = _make_rowdot(LROWS_PAD * EB, bt=2048)


def kernel(x, edge_index, edge_label_index, W1, b1, W2, b2):
    pad_e = EROWS_PAD * EB - E
    pad_src = (jnp.arange(pad_e, dtype=jnp.int32) * 61) % 4096
    pad_dst = N + (jnp.arange(pad_e, dtype=jnp.int32) % (NPAD - N))
    src2d = jnp.concatenate([edge_index[0], pad_src]).reshape(EROWS_PAD, EB)
    dst2d = jnp.concatenate([edge_index[1], pad_dst]).reshape(EROWS_PAD, EB)

    dga, dgb = _deg(dst2d)
    dinv, hs1c = _mm1(x, W1, dga[:N].reshape(N, 1), dgb[:N].reshape(N, 1))
    acc1 = _agg4(hs1c.reshape(4 * NPAD, 32), src2d, dst2d)
    hs2c = _epi1(acc1.reshape(4, NPAD, 32), hs1c, dinv,
                 b1.reshape(1, D1), W2)
    acc2 = _agg2(hs2c.reshape(2 * NPAD, 32), src2d, dst2d)
    z = _epi2(acc2.reshape(2, NPAD, 32), hs2c, dinv, b2.reshape(1, D2))

    pad_l = LROWS_PAD * EB - NLBL
    pad_i = (jnp.arange(pad_l, dtype=jnp.int32) * 61) % 4096
    ia2d = jnp.concatenate([edge_label_index[0], pad_i]).reshape(LROWS_PAD, EB)
    ib2d = jnp.concatenate([edge_label_index[1], pad_i]).reshape(LROWS_PAD, EB)
    za, zb = _decode(z, ia2d, ib2d)
    dots = _rowdot(za.reshape(-1, D2), zb.reshape(-1, D2))
    return dots[:NLBL, 0]


# speedup vs baseline: 27.8784x; 1.4508x over previous
import jax
import jax.numpy as jnp
from jax import lax
from jax.experimental import pallas as pl
from jax.experimental.pallas import tpu as pltpu
from jax.experimental.pallas import tpu_sc as plsc

N = 50000
E = 800000
NLBL = 100000
DIN, D1, D2 = 500, 128, 64

NC, NS = 2, 16
NW = NC * NS

EB = 128


def _cdiv(a, b):
    return (a + b - 1) // b


def _rup(a, b):
    return _cdiv(a, b) * b


NPAD = _rup(N, 128)
EROWS_PAD = _rup(_cdiv(E, EB), NW * 8)
LROWS_PAD = _rup(_cdiv(NLBL, EB), NW * 8)

_TRNG = 3200
_TLAST = NPAD - (NS - 1) * _TRNG


def _tile_rows(s):
    start = s * _TRNG
    nz = jnp.where(s < NS - 1, _TRNG // 128, _TLAST // 128)
    return start, nz


def _make_deg(db):
    rows_w = EROWS_PAD // NW
    nblk = rows_w // db

    def body(dst_hbm, out0_hbm, out1_hbm, dacc, dbuf, ones_v, zbuf):
        c = lax.axis_index("c")
        s = lax.axis_index("s")
        w = c * NS + s

        @pl.loop(0, 8)
        def _(k):
            ones_v[pl.ds(k * 16, 16)] = jnp.ones((16,), jnp.float32)

        @pl.loop(0, _TRNG // 16)
        def _(k):
            zbuf[pl.ds(k * 16, 16)] = jnp.zeros((16,), jnp.float32)

        start, nz = _tile_rows(s)

        @pl.when(s < NS - 1)
        def _():
            pltpu.sync_copy(zbuf, dacc.at[pl.ds(start, _TRNG)])

        @pl.when(s == NS - 1)
        def _():
            pltpu.sync_copy(zbuf.at[pl.ds(0, _TLAST)], dacc.at[pl.ds(start, _TLAST)])

        plsc.subcore_barrier()

        @pl.loop(0, nblk)
        def _(g):
            pltpu.sync_copy(dst_hbm.at[pl.ds(w * rows_w + g * db, db)], dbuf)
            for j in range(db):
                pltpu.sync_copy(ones_v, dacc.at[dbuf.at[j]], add=True)

        plsc.subcore_barrier()

        for cc, out_hbm in ((0, out0_hbm), (1, out1_hbm)):
            @pl.when(c == cc)
            def _():
                @pl.when(s < NS - 1)
                def _():
                    pltpu.sync_copy(dacc.at[pl.ds(start, _TRNG)],
                                    out_hbm.at[pl.ds(start, _TRNG)])

                @pl.when(s == NS - 1)
                def _():
                    pltpu.sync_copy(dacc.at[pl.ds(start, _TLAST)],
                                    out_hbm.at[pl.ds(start, _TLAST)])

    return pl.kernel(
        body,
        out_type=(jax.ShapeDtypeStruct((NPAD,), jnp.float32),
                  jax.ShapeDtypeStruct((NPAD,), jnp.float32)),
        mesh=plsc.VectorSubcoreMesh(core_axis_name="c", subcore_axis_name="s",
                                    num_cores=NC, num_subcores=NS),
        compiler_params=pltpu.CompilerParams(use_tc_tiling_on_sc=False),
        scratch_types=[
            pltpu.VMEM_SHARED((NPAD,), jnp.float32),
            pltpu.VMEM((db, EB), jnp.int32),
            pltpu.VMEM((EB,), jnp.float32),
            pltpu.VMEM((_TRNG,), jnp.float32),
        ],
    )


def _make_agg(nchunk):
    nr = nchunk // NC
    rows_t = EROWS_PAD // NS
    UNROLL = 8
    P = rows_t // UNROLL

    def body(table, src_hbm, dst_hbm, out_hbm, acc, sring, dring, oring, rbuf,
             zbuf, isem, gsem, ssem):
        c = lax.axis_index("c")
        s = lax.axis_index("s")

        @pl.loop(0, 64)
        def _(i):
            zbuf[i, pl.ds(0, 16)] = jnp.zeros((16,), jnp.float32)
            zbuf[i, pl.ds(16, 16)] = jnp.zeros((16,), jnp.float32)

        start, nz = _tile_rows(s)
        base = s * rows_t

        def issue_idx(row, v):
            pltpu.make_async_copy(src_hbm.at[base + row], sring.at[v],
                                  isem).start()
            pltpu.make_async_copy(dst_hbm.at[base + row], dring.at[v],
                                  isem).start()

        def wait_idx(v):
            for _ in range(2):
                pltpu.make_async_copy(src_hbm.at[base], sring.at[v],
                                      isem).wait()

        def wait_row(sem):
            pltpu.make_async_copy(table.at[pl.ds(0, 128)], rbuf.at[0],
                                  sem).wait()

        for r in range(nr):
            chunk = c * nr + r

            @pl.loop(0, nz * 2)
            def _(k):
                pltpu.sync_copy(zbuf, acc.at[pl.ds(start + k * 64, 64)])

            plsc.subcore_barrier()

            for v in range(4):
                issue_idx(v, v)

            @pl.loop(0, P)
            def _(p):
                for u in range(UNROLL):
                    g = UNROLL * p + u
                    s4, s8 = u % 4, u

                    def steady(u=u, s4=s4, s8=s8, g=g):
                        wait_row(ssem)

                    if u < 4:
                        pl.when(p > 0)(steady)
                    else:
                        steady()
                    wait_idx(s8)
                    for k in range(EB // 16):
                        oring[s4, pl.ds(k * 16, 16)] = (
                            sring[s8, pl.ds(k * 16, 16)] * nchunk + chunk)
                    pltpu.make_async_copy(table.at[oring.at[s4]],
                                          rbuf.at[s4], gsem).start()

                    def prefetch(g=g, v=(u + 4) % 8):
                        issue_idx(g + 4, v)

                    if u < 4:
                        prefetch()
                    else:
                        pl.when(p < P - 1)(prefetch)

                    def drain_gather(s4m2=(u - 2) % 4, s8m2=(u - 2) % 8):
                        wait_row(gsem)
                        pltpu.make_async_copy(
                            rbuf.at[s4m2], acc.at[dring.at[s8m2]],
                            ssem).start(add=True)

                    if u < 2:
                        pl.when(p > 0)(drain_gather)
                    else:
                        drain_gather()

            for u in (UNROLL - 2, UNROLL - 1):
                wait_row(gsem)
                pltpu.make_async_copy(rbuf.at[u % 4], acc.at[dring.at[u]],
                                      ssem).start(add=True)
            for _ in range(4):
                wait_row(ssem)

            plsc.subcore_barrier()

            @pl.loop(0, nz)
            def _(k):
                r0 = start + k * 128
                pltpu.sync_copy(acc.at[pl.ds(r0, 128)],
                                out_hbm.at[pl.ds(r0, 128),
                                           pl.ds(chunk * 32, 32)])

            plsc.subcore_barrier()

    return pl.kernel(
        body,
        out_type=jax.ShapeDtypeStruct((NPAD, 128), jnp.float32),
        mesh=plsc.VectorSubcoreMesh(core_axis_name="c", subcore_axis_name="s",
                                    num_cores=NC, num_subcores=NS),
        compiler_params=pltpu.CompilerParams(use_tc_tiling_on_sc=False),
        scratch_types=[
            pltpu.VMEM_SHARED((NPAD, 32), jnp.float32),
            pltpu.VMEM((8, EB), jnp.int32),
            pltpu.VMEM((8, EB), jnp.int32),
            pltpu.VMEM((4, EB), jnp.int32),
            pltpu.VMEM((4, EB, 32), jnp.float32),
            pltpu.VMEM((64, 32), jnp.float32),
            pltpu.SemaphoreType.DMA,
            pltpu.SemaphoreType.DMA,
            pltpu.SemaphoreType.DMA,
        ],
    )


def _make_decode(d, lblk):
    del lblk
    rows_w = LROWS_PAD // NW

    def body(z_hbm, ia_hbm, ib_hbm, zab_hbm, iabuf, ibbuf, rbuf,
             gsem, osem):
        c = lax.axis_index("c")
        s = lax.axis_index("s")
        w = c * NS + s
        b0 = w * rows_w

        pltpu.sync_copy(ia_hbm.at[pl.ds(b0, rows_w)], iabuf)
        pltpu.sync_copy(ib_hbm.at[pl.ds(b0, rows_w)], ibbuf)

        def wait_row(sem):
            pltpu.make_async_copy(z_hbm.at[pl.ds(0, EB)], rbuf.at[0, 0],
                                  sem).wait()

        def out_copy(h, row):
            pltpu.make_async_copy(
                rbuf.at[h, 0],
                zab_hbm.at[row].at[pl.ds(0, EB), pl.ds(0, D2)], osem).start()
            pltpu.make_async_copy(
                rbuf.at[h, 1],
                zab_hbm.at[row].at[pl.ds(0, EB), pl.ds(D2, D2)], osem).start()

        for g in range(rows_w):
            h = g % 2
            if g >= 2:
                wait_row(osem)
                wait_row(osem)
            pltpu.make_async_copy(z_hbm.at[iabuf.at[g]], rbuf.at[h, 0],
                                  gsem).start()
            pltpu.make_async_copy(z_hbm.at[ibbuf.at[g]], rbuf.at[h, 1],
                                  gsem).start()
            if g >= 1:
                wait_row(gsem)
                wait_row(gsem)
                out_copy(1 - h, b0 + g - 1)
        wait_row(gsem)
        wait_row(gsem)
        out_copy((rows_w - 1) % 2, b0 + rows_w - 1)
        for _ in range(4):
            wait_row(osem)

    return pl.kernel(
        body,
        out_type=jax.ShapeDtypeStruct((LROWS_PAD, EB, 2 * d), jnp.float32),
        mesh=plsc.VectorSubcoreMesh(core_axis_name="c", subcore_axis_name="s",
                                    num_cores=NC, num_subcores=NS),
        compiler_params=pltpu.CompilerParams(use_tc_tiling_on_sc=False),
        scratch_types=[
            pltpu.VMEM((LROWS_PAD // NW, EB), jnp.int32),
            pltpu.VMEM((LROWS_PAD // NW, EB), jnp.int32),
            pltpu.VMEM((2, 2, EB, d), jnp.float32),
            pltpu.SemaphoreType.DMA,
            pltpu.SemaphoreType.DMA,
        ],
    )


def _make_mm1(rb):
    nchunk = D1 // 32

    def body(x_ref, w_ref, dga_ref, dgb_ref, dinv_ref, hs_ref):
        h = jnp.dot(x_ref[...], w_ref[...], preferred_element_type=jnp.float32)
        d = dga_ref[...] + dgb_ref[...] + 1.0
        dinv = lax.rsqrt(d)
        dinv_ref[...] = dinv
        hs_ref[...] = h * dinv

    return pl.pallas_call(
        body,
        grid=(N // rb,),
        in_specs=[pl.BlockSpec((rb, DIN), lambda i: (i, 0)),
                  pl.BlockSpec((DIN, D1), lambda i: (0, 0)),
                  pl.BlockSpec((rb, 1), lambda i: (i, 0)),
                  pl.BlockSpec((rb, 1), lambda i: (i, 0))],
        out_specs=[pl.BlockSpec((rb, 1), lambda i: (i, 0)),
                   pl.BlockSpec((rb, D1), lambda i: (i, 0))],
        out_shape=[jax.ShapeDtypeStruct((N, 1), jnp.float32),
                   jax.ShapeDtypeStruct((NPAD, D1), jnp.float32)],
    )


def _make_layer1_epilogue(bt):
    def body(acc_ref, hs_ref, dinv_ref, b1_ref, w2_ref, hs2_ref):
        a = acc_ref[...] + hs_ref[...]
        dinv = dinv_ref[...]
        r1 = jnp.maximum(a * dinv + b1_ref[...], 0.0)
        h2 = jnp.dot(r1, w2_ref[...], preferred_element_type=jnp.float32)
        hs2_ref[...] = h2 * dinv

    return pl.pallas_call(
        body,
        grid=(N // bt,),
        in_specs=[pl.BlockSpec((bt, D1), lambda i: (i, 0)),
                  pl.BlockSpec((bt, D1), lambda i: (i, 0)),
                  pl.BlockSpec((bt, 1), lambda i: (i, 0)),
                  pl.BlockSpec((1, D1), lambda i: (0, 0)),
                  pl.BlockSpec((D1, D2), lambda i: (0, 0))],
        out_specs=pl.BlockSpec((bt, D2), lambda i: (i, 0)),
        out_shape=jax.ShapeDtypeStruct((NPAD, D2), jnp.float32),
    )


def _make_layer2_epilogue(bt):
    def body(acc_ref, hs2_ref, dinv_ref, b2_ref, z_ref):
        a = acc_ref[...][:, :D2] + hs2_ref[...]
        z_ref[...] = a * dinv_ref[...] + b2_ref[...]

    return pl.pallas_call(
        body,
        grid=(N // bt,),
        in_specs=[pl.BlockSpec((bt, 128), lambda i: (i, 0)),
                  pl.BlockSpec((bt, D2), lambda i: (i, 0)),
                  pl.BlockSpec((bt, 1), lambda i: (i, 0)),
                  pl.BlockSpec((1, D2), lambda i: (0, 0))],
        out_specs=pl.BlockSpec((bt, D2), lambda i: (i, 0)),
        out_shape=jax.ShapeDtypeStruct((N, D2), jnp.float32),
    )


def _make_rowdot(bt):
    def body(ab_ref, o_ref):
        ab = ab_ref[...]
        o_ref[...] = jnp.sum(ab[..., :D2] * ab[..., D2:], axis=-1)

    return pl.pallas_call(
        body,
        grid=(LROWS_PAD // bt,),
        in_specs=[pl.BlockSpec((bt, EB, 2 * D2), lambda i: (i, 0, 0))],
        out_specs=pl.BlockSpec((bt, EB), lambda i: (i, 0)),
        out_shape=jax.ShapeDtypeStruct((LROWS_PAD, EB), jnp.float32),
    )


_deg = _make_deg(db=8)
_agg4 = _make_agg(nchunk=4)
_agg2 = _make_agg(nchunk=2)
_decode = _make_decode(D2, lblk=8)
_mm1 = _make_mm1(rb=2000)
_epi1 = _make_layer1_epilogue(bt=2000)
_epi2 = _make_layer2_epilogue(bt=2000)
_rowdot = _make_rowdot(bt=32)


def kernel(x, edge_index, edge_label_index, W1, b1, W2, b2):
    pad_e = EROWS_PAD * EB - E
    pad_src = (jnp.arange(pad_e, dtype=jnp.int32) * 61) % 4096
    pad_dst = N + (jnp.arange(pad_e, dtype=jnp.int32) % (NPAD - N))
    src2d = jnp.concatenate([edge_index[0], pad_src]).reshape(EROWS_PAD, EB)
    dst2d = jnp.concatenate([edge_index[1], pad_dst]).reshape(EROWS_PAD, EB)

    dga, dgb = _deg(dst2d)
    dinv, hs = _mm1(x, W1, dga.reshape(NPAD, 1), dgb.reshape(NPAD, 1))
    tab1 = hs.reshape(4 * NPAD, 32)
    acc1 = _agg4(tab1, src2d, dst2d)
    hs2 = _epi1(acc1, hs, dinv, b1.reshape(1, D1), W2)
    tab2 = hs2.reshape(2 * NPAD, 32)
    acc2 = _agg2(tab2, src2d, dst2d)
    z = _epi2(acc2, hs2, dinv, b2.reshape(1, D2))

    pad_l = LROWS_PAD * EB - NLBL
    pad_i = (jnp.arange(pad_l, dtype=jnp.int32) * 61) % 4096
    ia2d = jnp.concatenate([edge_label_index[0], pad_i]).reshape(LROWS_PAD, EB)
    ib2d = jnp.concatenate([edge_label_index[1], pad_i]).reshape(LROWS_PAD, EB)
    zab = _decode(z, ia2d, ib2d)
    dots = _rowdot(zab)
    return dots.reshape(-1)[:NLBL]
